# baseline (speedup 1.0000x reference)
import functools

import jax
import jax.numpy as jnp
from jax import lax
from jax.experimental import pallas as pl
from jax.experimental.pallas import tpu as pltpu
from jax.experimental.pallas import tpu_sc as plsc

B = 16384
D = 64
W = 2 * D
NC, NS = 2, 16
NW = NC * NS
BPW = B // NW
CHUNK = 256
NCHUNK = BPW // CHUNK

BLK = 1024


def _sc_gather(emb_u2, emb_a2, emb_c2, lin_u1, lin_a1, lin_c1,
               pid_u, pid_a, pid_c, idx_u, idx_a, idx_c):
    mesh = plsc.VectorSubcoreMesh(core_axis_name="c", subcore_axis_name="s")
    f32 = jnp.float32
    out_types = (
        jax.ShapeDtypeStruct((B, W), f32),
        jax.ShapeDtypeStruct((B, W), f32),
        jax.ShapeDtypeStruct((B, W), f32),
        jax.ShapeDtypeStruct((B,), f32),
        jax.ShapeDtypeStruct((B,), f32),
        jax.ShapeDtypeStruct((B,), f32),
    )
    scratch = (
        [pltpu.VMEM((CHUNK,), jnp.int32) for _ in range(6)]
        + [pltpu.VMEM((CHUNK, W), f32) for _ in range(3)]
        + [pltpu.VMEM((CHUNK,), f32) for _ in range(3)]
        + [pltpu.SemaphoreType.DMA]
    )

    @functools.partial(pl.kernel, mesh=mesh, out_type=out_types,
                       scratch_types=scratch)
    def k(eu_h, ea_h, ec_h, lu_h, la_h, lc_h,
          pu_h, pa_h, pc_h, iu_h, ia_h, ic_h,
          oeu, oea, oec, olu, ola, olc,
          vpu, vpa, vpc, viu, via, vic,
          veu, vea, vec, vlu, vla, vlc, sem):
        wid = lax.axis_index("s") * NC + lax.axis_index("c")
        base = wid * BPW
        for c in range(NCHUNK):
            off = base + c * CHUNK
            pltpu.sync_copy(pu_h.at[pl.ds(off, CHUNK)], vpu)
            pltpu.sync_copy(pa_h.at[pl.ds(off, CHUNK)], vpa)
            pltpu.sync_copy(pc_h.at[pl.ds(off, CHUNK)], vpc)
            pltpu.sync_copy(iu_h.at[pl.ds(off, CHUNK)], viu)
            pltpu.sync_copy(ia_h.at[pl.ds(off, CHUNK)], via)
            pltpu.sync_copy(ic_h.at[pl.ds(off, CHUNK)], vic)
            h1 = pltpu.async_copy(eu_h.at[vpu], veu, sem)
            h2 = pltpu.async_copy(ea_h.at[vpa], vea, sem)
            h3 = pltpu.async_copy(ec_h.at[vpc], vec, sem)
            h4 = pltpu.async_copy(lu_h.at[viu], vlu, sem)
            h5 = pltpu.async_copy(la_h.at[via], vla, sem)
            h6 = pltpu.async_copy(lc_h.at[vic], vlc, sem)
            h1.wait()
            h2.wait()
            h3.wait()
            h4.wait()
            h5.wait()
            h6.wait()
            pltpu.sync_copy(veu, oeu.at[pl.ds(off, CHUNK)])
            pltpu.sync_copy(vea, oea.at[pl.ds(off, CHUNK)])
            pltpu.sync_copy(vec, oec.at[pl.ds(off, CHUNK)])
            pltpu.sync_copy(vlu, olu.at[pl.ds(off, CHUNK)])
            pltpu.sync_copy(vla, ola.at[pl.ds(off, CHUNK)])
            pltpu.sync_copy(vlc, olc.at[pl.ds(off, CHUNK)])

    return k(emb_u2, emb_a2, emb_c2, lin_u1, lin_a1, lin_c1,
             pid_u, pid_a, pid_c, idx_u, idx_a, idx_c)


def _tc_body(eu_r, ea_r, ec_r, lt_r, mu_r, ma_r, mc_r,
             W1_r, b1_r, W2_r, b2_r, w3_r, off_r, o_r):
    eu2 = eu_r[...]
    ea2 = ea_r[...]
    ec2 = ec_r[...]
    eu = jnp.where(mu_r[...] == 0, eu2[:, 0:D], eu2[:, D:W])
    ea = jnp.where(ma_r[...] == 0, ea2[:, 0:D], ea2[:, D:W])
    ec = jnp.where(mc_r[...] == 0, ec2[:, 0:D], ec2[:, D:W])
    s = eu + ea + ec
    sum_sq = jnp.sum(s * s, axis=1, keepdims=True)
    sq_sum = jnp.sum(eu * eu + ea * ea + ec * ec, axis=1, keepdims=True)
    interaction = 0.5 * (sum_sq - sq_sum)
    W1 = W1_r[...]
    f32 = jnp.float32
    h = (jnp.dot(eu, W1[0:D, :], preferred_element_type=f32)
         + jnp.dot(ea, W1[D:2 * D, :], preferred_element_type=f32)
         + jnp.dot(ec, W1[2 * D:3 * D, :], preferred_element_type=f32)
         + b1_r[...])
    h = jnp.maximum(h, 0.0)
    h = jnp.dot(h, W2_r[...], preferred_element_type=f32) + b2_r[...]
    h = jnp.maximum(h, 0.0)
    deep = jnp.sum(h * w3_r[...], axis=1, keepdims=True)
    o_r[...] = jax.nn.sigmoid(off_r[...] + lt_r[...] + interaction + deep)


def _tc_dense(eu, ea, ec, lt, mu, ma, mc, W1, b1, W2, b2, w3, off):
    grid = (B // BLK,)
    bs_emb = pl.BlockSpec((BLK, W), lambda i: (i, 0))
    bs_col = pl.BlockSpec((BLK, 1), lambda i: (i, 0))
    rep = lambda shape: pl.BlockSpec(shape, lambda i: (0, 0))
    return pl.pallas_call(
        _tc_body,
        grid=grid,
        in_specs=[
            bs_emb, bs_emb, bs_emb,
            bs_col, bs_col, bs_col, bs_col,
            rep((3 * D, 128)), rep((1, 128)),
            rep((128, D)), rep((1, D)),
            rep((1, D)), rep((1, 1)),
        ],
        out_specs=pl.BlockSpec((BLK, 1), lambda i: (i, 0)),
        out_shape=jax.ShapeDtypeStruct((B, 1), jnp.float32),
    )(eu, ea, ec, lt, mu, ma, mc, W1, b1, W2, b2, w3, off)


def kernel(user_id, ad_id, context_id, lin_user, lin_ad, lin_ctx,
           emb_user, emb_ad, emb_ctx, bias, W1, b1, W2, b2, W3, b3):
    eu2 = emb_user.reshape(-1, W)
    ea2 = emb_ad.reshape(-1, W)
    ec2 = emb_ctx.reshape(-1, W)
    lu1 = lin_user.reshape(-1)
    la1 = lin_ad.reshape(-1)
    lc1 = lin_ctx.reshape(-1)
    uid = user_id.astype(jnp.int32)
    aid = ad_id.astype(jnp.int32)
    cid = context_id.astype(jnp.int32)
    pu, pa, pc = uid // 2, aid // 2, cid // 2
    mu = (uid % 2).reshape(B, 1)
    ma = (aid % 2).reshape(B, 1)
    mc = (cid % 2).reshape(B, 1)

    geu, gea, gec, glu, gla, glc = _sc_gather(
        eu2, ea2, ec2, lu1, la1, lc1, pu, pa, pc, uid, aid, cid)

    lt = (glu + gla + glc).reshape(B, 1)
    off = (bias + b3).reshape(1, 1)
    out = _tc_dense(geu, gea, gec, lt, mu, ma, mc,
                    W1, b1.reshape(1, 128), W2, b2.reshape(1, D),
                    W3.reshape(1, D), off)
    return out.reshape(B)


# speedup vs baseline: 1.4172x; 1.4172x over previous
import functools

import jax
import jax.numpy as jnp
from jax import lax
from jax.experimental import pallas as pl
from jax.experimental.pallas import tpu as pltpu
from jax.experimental.pallas import tpu_sc as plsc

B = 16384
D = 64
NC, NS = 2, 16
NW = NC * NS
BPW = B // NW
CHUNK = 256
NCHUNK = BPW // CHUNK
G = 16

BLK = 1024


def _sc_gather(emb_u, emb_a, emb_c, lin_u1, lin_a1, lin_c1,
               idx_u, idx_a, idx_c):
    mesh = plsc.VectorSubcoreMesh(core_axis_name="c", subcore_axis_name="s")
    f32 = jnp.float32
    out_types = (
        jax.ShapeDtypeStruct((B, D), f32),
        jax.ShapeDtypeStruct((B, D), f32),
        jax.ShapeDtypeStruct((B, D), f32),
        jax.ShapeDtypeStruct((B,), f32),
        jax.ShapeDtypeStruct((B,), f32),
        jax.ShapeDtypeStruct((B,), f32),
    )
    scratch = (
        [pltpu.VMEM((CHUNK,), jnp.int32) for _ in range(3)]
        + [pltpu.VMEM((CHUNK, D), f32) for _ in range(3)]
        + [pltpu.VMEM((CHUNK,), f32) for _ in range(3)]
        + [pltpu.SemaphoreType.DMA, pltpu.SemaphoreType.DMA]
    )

    @functools.partial(pl.kernel, mesh=mesh, out_type=out_types,
                       scratch_types=scratch)
    def k(eu_h, ea_h, ec_h, lu_h, la_h, lc_h,
          iu_h, ia_h, ic_h,
          oeu, oea, oec, olu, ola, olc,
          viu, via, vic, veu, vea, vec, vlu, vla, vlc, sem, lsem):
        wid = lax.axis_index("s") * NC + lax.axis_index("c")
        base = wid * BPW
        for c in range(NCHUNK):
            off = base + c * CHUNK
            pltpu.sync_copy(iu_h.at[pl.ds(off, CHUNK)], viu)
            pltpu.sync_copy(ia_h.at[pl.ds(off, CHUNK)], via)
            pltpu.sync_copy(ic_h.at[pl.ds(off, CHUNK)], vic)
            hl = [pltpu.async_copy(lu_h.at[viu], vlu, lsem),
                  pltpu.async_copy(la_h.at[via], vla, lsem),
                  pltpu.async_copy(lc_h.at[vic], vlc, lsem)]

            @pl.loop(0, CHUNK // G)
            def _(g):
                vu = viu[pl.ds(g * G, G)]
                va = via[pl.ds(g * G, G)]
                vc = vic[pl.ds(g * G, G)]
                hs = []
                for j in range(G):
                    p = g * G + j
                    hs.append(pltpu.async_copy(
                        eu_h.at[pl.ds(vu[j], 1)], veu.at[pl.ds(p, 1)], sem))
                    hs.append(pltpu.async_copy(
                        ea_h.at[pl.ds(va[j], 1)], vea.at[pl.ds(p, 1)], sem))
                    hs.append(pltpu.async_copy(
                        ec_h.at[pl.ds(vc[j], 1)], vec.at[pl.ds(p, 1)], sem))
                for h in hs:
                    h.wait()

            for h in hl:
                h.wait()
            pltpu.sync_copy(veu, oeu.at[pl.ds(off, CHUNK)])
            pltpu.sync_copy(vea, oea.at[pl.ds(off, CHUNK)])
            pltpu.sync_copy(vec, oec.at[pl.ds(off, CHUNK)])
            pltpu.sync_copy(vlu, olu.at[pl.ds(off, CHUNK)])
            pltpu.sync_copy(vla, ola.at[pl.ds(off, CHUNK)])
            pltpu.sync_copy(vlc, olc.at[pl.ds(off, CHUNK)])

    return k(emb_u, emb_a, emb_c, lin_u1, lin_a1, lin_c1,
             idx_u, idx_a, idx_c)


def _tc_body(eu_r, ea_r, ec_r, lt_r,
             W1_r, b1_r, W2_r, b2_r, w3_r, off_r, o_r):
    eu = eu_r[...]
    ea = ea_r[...]
    ec = ec_r[...]
    s = eu + ea + ec
    sum_sq = jnp.sum(s * s, axis=1, keepdims=True)
    sq_sum = jnp.sum(eu * eu + ea * ea + ec * ec, axis=1, keepdims=True)
    interaction = 0.5 * (sum_sq - sq_sum)
    W1 = W1_r[...]
    f32 = jnp.float32
    h = (jnp.dot(eu, W1[0:D, :], preferred_element_type=f32)
         + jnp.dot(ea, W1[D:2 * D, :], preferred_element_type=f32)
         + jnp.dot(ec, W1[2 * D:3 * D, :], preferred_element_type=f32)
         + b1_r[...])
    h = jnp.maximum(h, 0.0)
    h = jnp.dot(h, W2_r[...], preferred_element_type=f32) + b2_r[...]
    h = jnp.maximum(h, 0.0)
    deep = jnp.sum(h * w3_r[...], axis=1, keepdims=True)
    o_r[...] = jax.nn.sigmoid(off_r[...] + lt_r[...] + interaction + deep)


def _tc_dense(eu, ea, ec, lt, W1, b1, W2, b2, w3, off):
    grid = (B // BLK,)
    bs_emb = pl.BlockSpec((BLK, D), lambda i: (i, 0))
    bs_col = pl.BlockSpec((BLK, 1), lambda i: (i, 0))
    rep = lambda shape: pl.BlockSpec(shape, lambda i: (0, 0))
    return pl.pallas_call(
        _tc_body,
        grid=grid,
        in_specs=[
            bs_emb, bs_emb, bs_emb, bs_col,
            rep((3 * D, 128)), rep((1, 128)),
            rep((128, D)), rep((1, D)),
            rep((1, D)), rep((1, 1)),
        ],
        out_specs=pl.BlockSpec((BLK, 1), lambda i: (i, 0)),
        out_shape=jax.ShapeDtypeStruct((B, 1), jnp.float32),
    )(eu, ea, ec, lt, W1, b1, W2, b2, w3, off)


def kernel(user_id, ad_id, context_id, lin_user, lin_ad, lin_ctx,
           emb_user, emb_ad, emb_ctx, bias, W1, b1, W2, b2, W3, b3):
    lu1 = lin_user.reshape(-1)
    la1 = lin_ad.reshape(-1)
    lc1 = lin_ctx.reshape(-1)
    uid = user_id.astype(jnp.int32)
    aid = ad_id.astype(jnp.int32)
    cid = context_id.astype(jnp.int32)

    geu, gea, gec, glu, gla, glc = _sc_gather(
        emb_user, emb_ad, emb_ctx, lu1, la1, lc1, uid, aid, cid)

    lt = (glu + gla + glc).reshape(B, 1)
    off = (bias + b3).reshape(1, 1)
    out = _tc_dense(geu, gea, gec, lt,
                    W1, b1.reshape(1, 128), W2, b2.reshape(1, D),
                    W3.reshape(1, D), off)
    return out.reshape(B)


# speedup vs baseline: 2.2324x; 1.5752x over previous
import functools

import jax
import jax.numpy as jnp
from jax import lax
from jax.experimental import pallas as pl
from jax.experimental.pallas import tpu as pltpu
from jax.experimental.pallas import tpu_sc as plsc

B = 16384
D = 64
NC, NS = 2, 16
NW = NC * NS
BPW = B // NW
CHUNK = 256
NCHUNK = BPW // CHUNK
G = 16
HALF = 500000

BLK = 1024


def _sc_gather(emb_u, emb_a, emb_c, lin_u1, lin_a1, lin_c1,
               idx_u, idx_a, idx_c):
    mesh = plsc.VectorSubcoreMesh(core_axis_name="c", subcore_axis_name="s")
    f32 = jnp.float32
    out_types = (
        jax.ShapeDtypeStruct((B, D), f32),
        jax.ShapeDtypeStruct((B, D), f32),
        jax.ShapeDtypeStruct((B, D), f32),
        jax.ShapeDtypeStruct((B,), f32),
        jax.ShapeDtypeStruct((B,), f32),
        jax.ShapeDtypeStruct((B,), f32),
    )
    scratch = (
        [pltpu.VMEM((CHUNK,), jnp.int32) for _ in range(3)]
        + [pltpu.VMEM((CHUNK, D), f32) for _ in range(3)]
        + [pltpu.VMEM((CHUNK,), f32) for _ in range(3)]
        + [pltpu.SemaphoreType.DMA, pltpu.SemaphoreType.DMA]
    )

    @functools.partial(pl.kernel, mesh=mesh, out_type=out_types,
                       scratch_types=scratch)
    def k(eu_h, ea_h, ec_h, lu_h, la_h, lc_h,
          iu_h, ia_h, ic_h,
          oeu, oea, oec, olu, ola, olc,
          viu, via, vic, veu, vea, vec, vlu, vla, vlc, sem, lsem):
        wid = lax.axis_index("s") * NC + lax.axis_index("c")
        base = wid * BPW
        for c in range(NCHUNK):
            off = base + c * CHUNK
            pltpu.sync_copy(iu_h.at[pl.ds(off, CHUNK)], viu)
            pltpu.sync_copy(ia_h.at[pl.ds(off, CHUNK)], via)
            pltpu.sync_copy(ic_h.at[pl.ds(off, CHUNK)], vic)
            hl = [pltpu.async_copy(lu_h.at[viu], vlu, lsem),
                  pltpu.async_copy(la_h.at[via], vla, lsem),
                  pltpu.async_copy(lc_h.at[vic], vlc, lsem)]

            @pl.loop(0, CHUNK // G)
            def _(g):
                vu = viu[pl.ds(g * G, G)]
                va = via[pl.ds(g * G, G)]
                vc = vic[pl.ds(g * G, G)]
                hs = []
                for j in range(G):
                    p = g * G + j
                    iu = vu[j]
                    hi = (iu >= HALF).astype(jnp.int32)
                    r2 = iu - hi * HALF
                    hs.append(pltpu.async_copy(
                        eu_h.at[hi].at[pl.ds(r2, 1)], veu.at[pl.ds(p, 1)],
                        sem))
                    hs.append(pltpu.async_copy(
                        ea_h.at[pl.ds(va[j], 1)], vea.at[pl.ds(p, 1)], sem))
                    hs.append(pltpu.async_copy(
                        ec_h.at[pl.ds(vc[j], 1)], vec.at[pl.ds(p, 1)], sem))
                for h in hs:
                    h.wait()

            for h in hl:
                h.wait()
            pltpu.sync_copy(veu, oeu.at[pl.ds(off, CHUNK)])
            pltpu.sync_copy(vea, oea.at[pl.ds(off, CHUNK)])
            pltpu.sync_copy(vec, oec.at[pl.ds(off, CHUNK)])
            pltpu.sync_copy(vlu, olu.at[pl.ds(off, CHUNK)])
            pltpu.sync_copy(vla, ola.at[pl.ds(off, CHUNK)])
            pltpu.sync_copy(vlc, olc.at[pl.ds(off, CHUNK)])

    return k(emb_u, emb_a, emb_c, lin_u1, lin_a1, lin_c1,
             idx_u, idx_a, idx_c)


def _tc_body(eu_r, ea_r, ec_r, lt_r,
             W1_r, b1_r, W2_r, b2_r, w3_r, off_r, o_r):
    eu = eu_r[...]
    ea = ea_r[...]
    ec = ec_r[...]
    s = eu + ea + ec
    sum_sq = jnp.sum(s * s, axis=1, keepdims=True)
    sq_sum = jnp.sum(eu * eu + ea * ea + ec * ec, axis=1, keepdims=True)
    interaction = 0.5 * (sum_sq - sq_sum)
    W1 = W1_r[...]
    f32 = jnp.float32
    h = (jnp.dot(eu, W1[0:D, :], preferred_element_type=f32)
         + jnp.dot(ea, W1[D:2 * D, :], preferred_element_type=f32)
         + jnp.dot(ec, W1[2 * D:3 * D, :], preferred_element_type=f32)
         + b1_r[...])
    h = jnp.maximum(h, 0.0)
    h = jnp.dot(h, W2_r[...], preferred_element_type=f32) + b2_r[...]
    h = jnp.maximum(h, 0.0)
    deep = jnp.sum(h * w3_r[...], axis=1, keepdims=True)
    o_r[...] = jax.nn.sigmoid(off_r[...] + lt_r[...] + interaction + deep)


def _tc_dense(eu, ea, ec, lt, W1, b1, W2, b2, w3, off):
    grid = (B // BLK,)
    bs_emb = pl.BlockSpec((BLK, D), lambda i: (i, 0))
    bs_col = pl.BlockSpec((BLK, 1), lambda i: (i, 0))
    rep = lambda shape: pl.BlockSpec(shape, lambda i: (0, 0))
    return pl.pallas_call(
        _tc_body,
        grid=grid,
        in_specs=[
            bs_emb, bs_emb, bs_emb, bs_col,
            rep((3 * D, 128)), rep((1, 128)),
            rep((128, D)), rep((1, D)),
            rep((1, D)), rep((1, 1)),
        ],
        out_specs=pl.BlockSpec((BLK, 1), lambda i: (i, 0)),
        out_shape=jax.ShapeDtypeStruct((B, 1), jnp.float32),
    )(eu, ea, ec, lt, W1, b1, W2, b2, w3, off)


def kernel(user_id, ad_id, context_id, lin_user, lin_ad, lin_ctx,
           emb_user, emb_ad, emb_ctx, bias, W1, b1, W2, b2, W3, b3):
    lu1 = lin_user.reshape(-1)
    la1 = lin_ad.reshape(-1)
    lc1 = lin_ctx.reshape(-1)
    uid = user_id.astype(jnp.int32)
    aid = ad_id.astype(jnp.int32)
    cid = context_id.astype(jnp.int32)

    u3 = emb_user.reshape(2, HALF, D)

    geu, gea, gec, glu, gla, glc = _sc_gather(
        u3, emb_ad, emb_ctx, lu1, la1, lc1, uid, aid, cid)

    lt = (glu + gla + glc).reshape(B, 1)
    off = (bias + b3).reshape(1, 1)
    out = _tc_dense(geu, gea, gec, lt,
                    W1, b1.reshape(1, 128), W2, b2.reshape(1, D),
                    W3.reshape(1, D), off)
    return out.reshape(B)
